# baseline (speedup 1.0000x reference)
import functools

import jax
import jax.numpy as jnp
from jax.experimental import pallas as pl
from jax.experimental.pallas import tpu as pltpu


def _round_up(x, m):
    return ((x + m - 1) // m) * m


def _head_kernel(x_ref, w1t_ref, b1_ref, w2t_ref, b2_ref, out_ref, acc_ref,
                 *, hw, thw, inv_hw):
    k = pl.program_id(1)

    @pl.when(k == 0)
    def _():
        acc_ref[...] = jnp.zeros_like(acc_ref)

    x = x_ref[...]
    if hw % thw:
        lane = jax.lax.broadcasted_iota(jnp.int32, x.shape, 2)
        x = jnp.where(lane + k * thw < hw, x, 0.0)

    chunks = [x[:, :, s * 128:(s + 1) * 128] for s in range(thw // 128)]
    while len(chunks) > 1:
        nxt = [chunks[i] + chunks[i + 1] for i in range(0, len(chunks) - 1, 2)]
        if len(chunks) % 2:
            nxt.append(chunks[-1])
        chunks = nxt
    acc_ref[...] += chunks[0]

    @pl.when(k == pl.num_programs(1) - 1)
    def _():
        pooled = jnp.sum(acc_ref[...], axis=-1) * inv_hw
        h = jnp.dot(pooled, w1t_ref[...],
                    preferred_element_type=jnp.float32) + b1_ref[...]
        out = jnp.dot(h, w2t_ref[...],
                      preferred_element_type=jnp.float32) + b2_ref[...]
        out_ref[...] = out.astype(out_ref.dtype)


def kernel(x, w1, b1, w2, b2):
    B, C, H, W = x.shape
    hidden = w1.shape[0]
    NC = w2.shape[0]
    HW = H * W

    TB = 8
    THW = min(512, _round_up(HW, 128))
    B_pad = _round_up(max(B, TB), TB)
    H_pad = _round_up(hidden, 128)
    NC_pad = _round_up(NC, 128)

    xr = x.reshape(B, C, HW)
    if B_pad != B:
        xr = jnp.pad(xr, ((0, B_pad - B), (0, 0), (0, 0)))

    w1t = jnp.pad(w1.T, ((0, 0), (0, H_pad - hidden)))
    b1_row = jnp.pad(b1.reshape(1, -1), ((0, 0), (0, H_pad - hidden)))
    w2t = jnp.pad(w2.T, ((0, H_pad - hidden), (0, NC_pad - NC)))
    b2_row = jnp.pad(b2.reshape(1, -1), ((0, 0), (0, NC_pad - NC)))

    n_b = B_pad // TB
    n_hw = -(-HW // THW)
    x_tile_bytes = TB * C * THW * 4
    weight_bytes = (C * H_pad + H_pad + H_pad * NC_pad + NC_pad) * 4
    vmem_limit = min(3 * x_tile_bytes + 2 * weight_bytes
                     + TB * C * 128 * 4 + TB * NC_pad * 4 + (8 << 20),
                     100 << 20)

    cost = pl.CostEstimate(
        flops=B_pad * C * HW + 2 * B_pad * (C * H_pad + H_pad * NC_pad),
        transcendentals=0,
        bytes_accessed=(B_pad * C * HW * 4 + weight_bytes + B_pad * NC_pad * 4),
    )

    out_padded = pl.pallas_call(
        functools.partial(_head_kernel, hw=HW, thw=THW, inv_hw=1.0 / float(HW)),
        out_shape=jax.ShapeDtypeStruct((B_pad, NC_pad), jnp.float32),
        grid_spec=pltpu.PrefetchScalarGridSpec(
            num_scalar_prefetch=0,
            grid=(n_b, n_hw),
            in_specs=[
                pl.BlockSpec((TB, C, THW), lambda i, k: (i, 0, k)),
                pl.BlockSpec((C, H_pad), lambda i, k: (0, 0)),
                pl.BlockSpec((1, H_pad), lambda i, k: (0, 0)),
                pl.BlockSpec((H_pad, NC_pad), lambda i, k: (0, 0)),
                pl.BlockSpec((1, NC_pad), lambda i, k: (0, 0)),
            ],
            out_specs=pl.BlockSpec((TB, NC_pad), lambda i, k: (i, 0)),
            scratch_shapes=[pltpu.VMEM((TB, C, 128), jnp.float32)],
        ),
        compiler_params=pltpu.CompilerParams(
            dimension_semantics=("parallel", "arbitrary"),
            vmem_limit_bytes=vmem_limit,
        ),
        cost_estimate=cost,
    )(xr, w1t, b1_row, w2t, b2_row)

    return out_padded[:B, :NC]
